# baseline (speedup 1.0000x reference)
import functools

import jax
import jax.numpy as jnp
from jax import lax
from jax.experimental import pallas as pl
from jax.experimental.pallas import tpu as pltpu
from jax.experimental.pallas import tpu_sc as plsc

M = 1_000_000
DIM = 64
LANES = 16
NW = 32
TILE = 256
FULL_TILES = M // TILE
TAIL = M - FULL_TILES * TILE
ITERS = (FULL_TILES + NW - 1) // NW


def _sc_tile(x_ref, w_ref, out_ref, buf, obuf, width, dst_base):
    nk = width // LANES
    acc = [jnp.zeros((LANES,), jnp.float32) for _ in range(nk)]
    for d in range(DIM):
        wd = w_ref[d]
        for k in range(nk):
            acc[k] = acc[k] + buf[d, pl.ds(k * LANES, LANES)] * wd
    for k in range(nk):
        obuf[pl.ds(k * LANES, LANES)] = acc[k]
    pltpu.sync_copy(obuf, out_ref.at[pl.ds(dst_base, width)])


def _sc_body(x_ref, w_hbm, out_ref, buf, obuf, tbuf, tobuf, w_ref):
    wid = lax.axis_index("s") * 2 + lax.axis_index("c")
    pltpu.sync_copy(w_hbm, w_ref)

    def step(i, _):
        t = wid + i * NW

        @pl.when(t < FULL_TILES)
        def _():
            base = t * TILE
            pltpu.sync_copy(x_ref.at[:, pl.ds(base, TILE)], buf)
            _sc_tile(x_ref, w_ref, out_ref, buf, obuf, TILE, base)

        return 0

    lax.fori_loop(0, ITERS, step, 0)

    @pl.when(wid == 2)
    def _():
        base = FULL_TILES * TILE
        pltpu.sync_copy(x_ref.at[:, pl.ds(base, TAIL)], tbuf)
        _sc_tile(x_ref, w_ref, out_ref, tbuf, tobuf, TAIL, base)


@functools.partial(jax.jit, static_argnums=())
def _sc_scores(xt, w_bcast):
    mesh = plsc.VectorSubcoreMesh(core_axis_name="c", subcore_axis_name="s")
    return pl.kernel(
        _sc_body,
        mesh=mesh,
        out_type=jax.ShapeDtypeStruct((M,), jnp.float32),
        scratch_types=[
            pltpu.VMEM((DIM, TILE), jnp.float32),
            pltpu.VMEM((TILE,), jnp.float32),
            pltpu.VMEM((DIM, TAIL), jnp.float32),
            pltpu.VMEM((TAIL,), jnp.float32),
            pltpu.VMEM((DIM, LANES), jnp.float32),
        ],
    )(xt, w_bcast)


def kernel(items_emb, user_w):
    m, dim = items_emb.shape
    xt = items_emb.T
    w_bcast = jnp.tile(user_w.reshape(dim, 1), (1, LANES))
    return _sc_scores(xt, w_bcast)


# speedup vs baseline: 1.2796x; 1.2796x over previous
import functools

import jax
import jax.numpy as jnp
from jax import lax
from jax.experimental import pallas as pl
from jax.experimental.pallas import tpu as pltpu
from jax.experimental.pallas import tpu_sc as plsc

M = 1_000_000
DIM = 64
LANES = 16
NW = 32
TILE = 256
FULL_TILES = M // TILE
TAIL = M - FULL_TILES * TILE
PAIRS = 62


def _acc_tile(w_ref, buf, obuf, width):
    nk = width // LANES
    acc = [jnp.zeros((LANES,), jnp.float32) for _ in range(nk)]
    for d in range(DIM):
        wd = w_ref[d]
        for k in range(nk):
            acc[k] = acc[k] + buf[d, pl.ds(k * LANES, LANES)] * wd
    for k in range(nk):
        obuf[pl.ds(k * LANES, LANES)] = acc[k]


def _sc_body(x_ref, w_hbm, out_ref,
             buf0, buf1, ob0, ob1, tbuf, tobuf, w_ref,
             si0, si1, so0, so1):
    wid = lax.axis_index("s") * 2 + lax.axis_index("c")
    pltpu.sync_copy(w_hbm, w_ref)

    bufs = (buf0, buf1)
    obufs = (ob0, ob1)
    sin = (si0, si1)
    sout = (so0, so1)

    def tile_base(i):
        return jnp.minimum(wid + i * NW, FULL_TILES - 1) * TILE

    def start_in(i, b):
        pltpu.make_async_copy(
            x_ref.at[:, pl.ds(tile_base(i), TILE)], bufs[b], sin[b]
        ).start()

    def wait_in(b):
        pltpu.make_async_copy(
            x_ref.at[:, pl.ds(0, TILE)], bufs[b], sin[b]
        ).wait()

    def start_out(i, b):
        pltpu.make_async_copy(
            obufs[b], out_ref.at[pl.ds(tile_base(i), TILE)], sout[b]
        ).start()

    def wait_out(b):
        pltpu.make_async_copy(
            obufs[b], out_ref.at[pl.ds(0, TILE)], sout[b]
        ).wait()

    start_in(0, 0)

    def step(j, _):
        i0 = j * 2
        i1 = i0 + 1

        wait_in(0)
        start_in(i1, 1)

        @pl.when(j > 0)
        def _():
            wait_out(0)

        _acc_tile(w_ref, buf0, ob0, TILE)
        start_out(i0, 0)

        wait_in(1)

        @pl.when(i1 + 1 < 2 * PAIRS)
        def _():
            start_in(i1 + 1, 0)

        @pl.when(j > 0)
        def _():
            wait_out(1)

        _acc_tile(w_ref, buf1, ob1, TILE)
        start_out(i1, 1)
        return 0

    lax.fori_loop(0, PAIRS, step, 0)
    wait_out(0)
    wait_out(1)

    @pl.when(wid == 2)
    def _():
        base = FULL_TILES * TILE
        pltpu.sync_copy(x_ref.at[:, pl.ds(base, TAIL)], tbuf)
        _acc_tile(w_ref, tbuf, tobuf, TAIL)
        pltpu.sync_copy(tobuf, out_ref.at[pl.ds(base, TAIL)])


@functools.partial(jax.jit, static_argnums=())
def _sc_scores(xt, w_bcast):
    mesh = plsc.VectorSubcoreMesh(core_axis_name="c", subcore_axis_name="s")
    return pl.kernel(
        _sc_body,
        mesh=mesh,
        out_type=jax.ShapeDtypeStruct((M,), jnp.float32),
        scratch_types=[
            pltpu.VMEM((DIM, TILE), jnp.float32),
            pltpu.VMEM((DIM, TILE), jnp.float32),
            pltpu.VMEM((TILE,), jnp.float32),
            pltpu.VMEM((TILE,), jnp.float32),
            pltpu.VMEM((DIM, TAIL), jnp.float32),
            pltpu.VMEM((TAIL,), jnp.float32),
            pltpu.VMEM((DIM, LANES), jnp.float32),
            pltpu.SemaphoreType.DMA,
            pltpu.SemaphoreType.DMA,
            pltpu.SemaphoreType.DMA,
            pltpu.SemaphoreType.DMA,
        ],
    )(xt, w_bcast)


def kernel(items_emb, user_w):
    m, dim = items_emb.shape
    xt = items_emb.T
    w_bcast = jnp.tile(user_w.reshape(dim, 1), (1, LANES))
    return _sc_scores(xt, w_bcast)


# speedup vs baseline: 4.9995x; 3.9070x over previous
import functools

import jax
import jax.numpy as jnp
from jax import lax
from jax.experimental import pallas as pl
from jax.experimental.pallas import tpu as pltpu
from jax.experimental.pallas import tpu_sc as plsc

M = 1_000_000
DIM = 64
LANES = 16
NW = 32
TILE = 256
PAIRS = 10
SC_ELEMS = NW * TILE * 2 * PAIRS
TC_BLK = 16384
SC_BLKS = SC_ELEMS // TC_BLK
TC_ELEMS = M - SC_ELEMS
TC_GRID = (TC_ELEMS + TC_BLK - 1) // TC_BLK


def _acc_tile(w_ref, buf, obuf, width):
    nk = width // LANES
    acc = [jnp.zeros((LANES,), jnp.float32) for _ in range(nk)]
    for d in range(DIM):
        wd = w_ref[d]
        for k in range(nk):
            acc[k] = acc[k] + buf[d, pl.ds(k * LANES, LANES)] * wd
    for k in range(nk):
        obuf[pl.ds(k * LANES, LANES)] = acc[k]


def _sc_body(x_ref, w_hbm, out_ref,
             buf0, buf1, ob0, ob1, w_ref,
             si0, si1, so0, so1):
    wid = lax.axis_index("s") * 2 + lax.axis_index("c")
    pltpu.sync_copy(w_hbm, w_ref)

    bufs = (buf0, buf1)
    obufs = (ob0, ob1)
    sin = (si0, si1)
    sout = (so0, so1)

    def tile_base(i):
        return (wid + i * NW) * TILE

    def start_in(i, b):
        pltpu.make_async_copy(
            x_ref.at[:, pl.ds(tile_base(i), TILE)], bufs[b], sin[b]
        ).start()

    def wait_in(b):
        pltpu.make_async_copy(
            x_ref.at[:, pl.ds(0, TILE)], bufs[b], sin[b]
        ).wait()

    def start_out(i, b):
        pltpu.make_async_copy(
            obufs[b], out_ref.at[pl.ds(tile_base(i), TILE)], sout[b]
        ).start()

    def wait_out(b):
        pltpu.make_async_copy(
            obufs[b], out_ref.at[pl.ds(0, TILE)], sout[b]
        ).wait()

    start_in(0, 0)

    def step(j, _):
        i0 = j * 2
        i1 = i0 + 1

        wait_in(0)
        start_in(i1, 1)

        @pl.when(j > 0)
        def _():
            wait_out(0)

        _acc_tile(w_ref, buf0, ob0, TILE)
        start_out(i0, 0)

        wait_in(1)

        @pl.when(i1 + 1 < 2 * PAIRS)
        def _():
            start_in(i1 + 1, 0)

        @pl.when(j > 0)
        def _():
            wait_out(1)

        _acc_tile(w_ref, buf1, ob1, TILE)
        start_out(i1, 1)
        return 0

    lax.fori_loop(0, PAIRS, step, 0)
    wait_out(0)
    wait_out(1)


def _sc_scores(xt, w_bcast):
    mesh = plsc.VectorSubcoreMesh(core_axis_name="c", subcore_axis_name="s")
    return pl.kernel(
        _sc_body,
        mesh=mesh,
        out_type=jax.ShapeDtypeStruct((SC_ELEMS,), jnp.float32),
        scratch_types=[
            pltpu.VMEM((DIM, TILE), jnp.float32),
            pltpu.VMEM((DIM, TILE), jnp.float32),
            pltpu.VMEM((TILE,), jnp.float32),
            pltpu.VMEM((TILE,), jnp.float32),
            pltpu.VMEM((DIM, LANES), jnp.float32),
            pltpu.SemaphoreType.DMA,
            pltpu.SemaphoreType.DMA,
            pltpu.SemaphoreType.DMA,
            pltpu.SemaphoreType.DMA,
        ],
    )(xt, w_bcast)


def _tc_body(w_ref, x_ref, o_ref):
    o_ref[...] = jnp.sum(x_ref[...] * w_ref[...], axis=0)


def _tc_scores(xt, w_col):
    return pl.pallas_call(
        _tc_body,
        grid=(TC_GRID,),
        in_specs=[
            pl.BlockSpec((DIM, 1), lambda i: (0, 0)),
            pl.BlockSpec((DIM, TC_BLK), lambda i: (0, i + SC_BLKS)),
        ],
        out_specs=pl.BlockSpec((TC_BLK,), lambda i: (i,)),
        out_shape=jax.ShapeDtypeStruct((TC_ELEMS,), jnp.float32),
    )(w_col, xt)


def kernel(items_emb, user_w):
    m, dim = items_emb.shape
    xt = items_emb.T
    w_bcast = jnp.tile(user_w.reshape(dim, 1), (1, LANES))
    sc_out = _sc_scores(xt, w_bcast)
    tc_out = _tc_scores(xt, user_w.reshape(dim, 1))
    return jnp.concatenate([sc_out, tc_out], axis=0)
